# baseline (speedup 1.0000x reference)
import functools

import jax
import jax.numpy as jnp
import numpy as np
from jax import lax
from jax.experimental import pallas as pl
from jax.experimental.pallas import tpu as pltpu
from jax.experimental.pallas import tpu_sc as plsc

B, BLOCKS, H, K2 = 32, 4096, 8, 49
SIZE = 7
POS = 2 * SIZE - 1
N = BLOCKS * H * K2


def _pad_bias_np(indexes, total, dilation_rate):
    size = indexes.shape[0]
    left = np.repeat(indexes[: size // 2], dilation_rate)
    right = np.repeat(indexes[size // 2 + 1:], dilation_rate)
    center = np.repeat(indexes[size // 2], total - left.shape[0] - right.shape[0])
    return np.concatenate([left, center, right], axis=-1)


@functools.lru_cache(maxsize=1)
def _bias_hw_rev():
    height = int(np.sqrt(float(BLOCKS)))
    width = BLOCKS // height
    idx_hh = np.arange(0, SIZE)
    idx_ww = np.arange(0, SIZE)
    bias_hh = _pad_bias_np(idx_hh, total=height, dilation_rate=1)
    bias_ww = _pad_bias_np(idx_ww, total=width, dilation_rate=1)
    bias_hw = (np.expand_dims(bias_hh, -1) * POS + bias_ww).reshape(-1)[::-1]
    return np.ascontiguousarray(bias_hw.astype(np.int32))


ROWS = K2 * H
GROUP = 8
NGROUPS = ROWS // GROUP


def _build_bias_sc(pos_bias_flat, bias_hw):
    info = plsc.get_sparse_core_info()
    nc, ns, lanes = info.num_cores, info.num_subcores, info.num_lanes
    nw = nc * ns
    stripe = BLOCKS // nw
    table_words = POS * POS * H

    mesh = plsc.VectorSubcoreMesh(core_axis_name="c", subcore_axis_name="s")

    @functools.partial(
        pl.kernel,
        mesh=mesh,
        compiler_params=pltpu.CompilerParams(needs_layout_passes=False),
        out_type=jax.ShapeDtypeStruct((ROWS, BLOCKS), jnp.float32),
        scratch_types=[
            pltpu.VMEM((table_words,), jnp.float32),
            pltpu.VMEM((stripe,), jnp.int32),
            pltpu.VMEM((ROWS, stripe), jnp.float32),
        ],
    )
    def gather_kernel(table_hbm, hw_hbm, out_hbm, tab_v, hw_v, val_v):
        wid = lax.axis_index("s") * nc + lax.axis_index("c")
        col0 = pl.multiple_of(wid * stripe, stripe)
        pltpu.sync_copy(table_hbm, tab_v)
        pltpu.sync_copy(hw_hbm.at[pl.ds(col0, stripe)], hw_v)

        hw_chunks = [hw_v[pl.ds(j * lanes, lanes)] for j in range(stripe // lanes)]

        def k_body(kk, carry):
            coords_k = (kk // SIZE) * POS + kk % SIZE
            for h in range(H):
                base = coords_k + h * (POS * POS)
                r = kk * H + h
                for j, hw_c in enumerate(hw_chunks):
                    idx = hw_c + base
                    val_v[r, pl.ds(j * lanes, lanes)] = plsc.load_gather(tab_v, [idx])
            return carry

        lax.fori_loop(0, K2, k_body, 0)
        pltpu.sync_copy(val_v, out_hbm.at[:, pl.ds(col0, stripe)])

    return gather_kernel(pos_bias_flat, bias_hw)


def _add_bias_tc(x_t, bias_t, c=4096):
    grid = (BLOCKS // c, B)

    def add_body(x_ref, b_ref, o_ref):
        o_ref[0] = x_ref[0] + b_ref[...]

    return pl.pallas_call(
        add_body,
        grid=grid,
        in_specs=[
            pl.BlockSpec((1, K2, H, c), lambda j, b: (b, 0, 0, j)),
            pl.BlockSpec((K2, H, c), lambda j, b: (0, 0, j)),
        ],
        out_specs=pl.BlockSpec((1, K2, H, c), lambda j, b: (b, 0, 0, j)),
        out_shape=jax.ShapeDtypeStruct((B, K2, H, BLOCKS), jnp.float32),
    )(x_t, bias_t)


def kernel(inputs, pos_bias):
    bias_hw = jnp.asarray(_bias_hw_rev())
    bias2d = _build_bias_sc(jnp.reshape(pos_bias, (-1,)), bias_hw)
    bias_t = jnp.reshape(bias2d, (K2, H, BLOCKS))
    x_t = jnp.transpose(inputs, (0, 3, 2, 1))
    out_t = _add_bias_tc(x_t, bias_t)
    return jnp.transpose(out_t, (0, 3, 2, 1))


# speedup vs baseline: 1.0859x; 1.0859x over previous
import functools

import jax
import jax.numpy as jnp
import numpy as np
from jax import lax
from jax.experimental import pallas as pl
from jax.experimental.pallas import tpu as pltpu
from jax.experimental.pallas import tpu_sc as plsc

B, BLOCKS, H, K2 = 32, 4096, 8, 49
SIZE = 7
POS = 2 * SIZE - 1
N = BLOCKS * H * K2


def _pad_bias_np(indexes, total, dilation_rate):
    size = indexes.shape[0]
    left = np.repeat(indexes[: size // 2], dilation_rate)
    right = np.repeat(indexes[size // 2 + 1:], dilation_rate)
    center = np.repeat(indexes[size // 2], total - left.shape[0] - right.shape[0])
    return np.concatenate([left, center, right], axis=-1)


@functools.lru_cache(maxsize=1)
def _bias_hw_rev():
    height = int(np.sqrt(float(BLOCKS)))
    width = BLOCKS // height
    idx_hh = np.arange(0, SIZE)
    idx_ww = np.arange(0, SIZE)
    bias_hh = _pad_bias_np(idx_hh, total=height, dilation_rate=1)
    bias_ww = _pad_bias_np(idx_ww, total=width, dilation_rate=1)
    bias_hw = (np.expand_dims(bias_hh, -1) * POS + bias_ww).reshape(-1)[::-1]
    return np.ascontiguousarray(bias_hw.astype(np.int32))


ROWS = K2 * H
GROUP = 8
NGROUPS = ROWS // GROUP


def _build_bias_sc(pos_bias_flat, bias_hw):
    info = plsc.get_sparse_core_info()
    nc, ns, lanes = info.num_cores, info.num_subcores, info.num_lanes
    nw = nc * ns
    stripe = BLOCKS // nw
    table_words = POS * POS * H

    mesh = plsc.VectorSubcoreMesh(core_axis_name="c", subcore_axis_name="s")

    @functools.partial(
        pl.kernel,
        mesh=mesh,
        compiler_params=pltpu.CompilerParams(needs_layout_passes=False),
        out_type=jax.ShapeDtypeStruct((ROWS, BLOCKS), jnp.float32),
        scratch_types=[
            pltpu.VMEM((table_words,), jnp.float32),
            pltpu.VMEM((stripe,), jnp.int32),
            pltpu.VMEM((ROWS, stripe), jnp.float32),
        ],
    )
    def gather_kernel(table_hbm, hw_hbm, out_hbm, tab_v, hw_v, val_v):
        wid = lax.axis_index("s") * nc + lax.axis_index("c")
        col0 = pl.multiple_of(wid * stripe, stripe)
        pltpu.sync_copy(table_hbm, tab_v)
        pltpu.sync_copy(hw_hbm.at[pl.ds(col0, stripe)], hw_v)

        hw_chunks = [hw_v[pl.ds(j * lanes, lanes)] for j in range(stripe // lanes)]

        @functools.partial(plsc.parallel_loop, 0, K2)
        def k_body(kk):
            coords_k = (kk // SIZE) * POS + kk % SIZE
            for h in range(H):
                base = coords_k + h * (POS * POS)
                r = kk * H + h
                for j, hw_c in enumerate(hw_chunks):
                    idx = hw_c + base
                    val_v[r, pl.ds(j * lanes, lanes)] = plsc.load_gather(tab_v, [idx])

        plsc.subcore_barrier()
        pltpu.sync_copy(val_v, out_hbm.at[:, pl.ds(col0, stripe)])

    return gather_kernel(pos_bias_flat, bias_hw)


def _add_bias_tc(x_t, bias_t, c=4096):
    grid = (BLOCKS // c, B)

    def add_body(x_ref, b_ref, o_ref):
        o_ref[0] = x_ref[0] + b_ref[...]

    return pl.pallas_call(
        add_body,
        grid=grid,
        in_specs=[
            pl.BlockSpec((1, K2, H, c), lambda j, b: (b, 0, 0, j)),
            pl.BlockSpec((K2, H, c), lambda j, b: (0, 0, j)),
        ],
        out_specs=pl.BlockSpec((1, K2, H, c), lambda j, b: (b, 0, 0, j)),
        out_shape=jax.ShapeDtypeStruct((B, K2, H, BLOCKS), jnp.float32),
    )(x_t, bias_t)


def kernel(inputs, pos_bias):
    bias_hw = jnp.asarray(_bias_hw_rev())
    bias2d = _build_bias_sc(jnp.reshape(pos_bias, (-1,)), bias_hw)
    bias_t = jnp.reshape(bias2d, (K2, H, BLOCKS))
    x_t = jnp.transpose(inputs, (0, 3, 2, 1))
    out_t = _add_bias_tc(x_t, bias_t)
    return jnp.transpose(out_t, (0, 3, 2, 1))
